# baseline (speedup 1.0000x reference)
import jax
import jax.numpy as jnp
from jax import lax
from jax.experimental import pallas as pl
from jax.experimental.pallas import tpu as pltpu
from jax.experimental.pallas import tpu_sc as plsc

NUM_CLASSES = 1000
FEATURE_DIM = 512
BATCH = 16384

NCORES = 2
NUM_WORKERS = 32
ROWS_PER_WORKER = BATCH // NUM_WORKERS
CHUNK = 64
NCHUNKS = ROWS_PER_WORKER // CHUNK
ACC_ROWS = 1024
AUX_W = 128
KSPLIT = FEATURE_DIM // 128


def _sc_phase1(input_hbm, tgt_hbm, out_m_hbm, out_a_hbm,
               xb00, xb01, xb02, xb03, xb10, xb11, xb12, xb13,
               aux0, aux1, tgt, m_sh0, m_sh1, m_sh2, m_sh3, a_sh,
               isem0, isem1, xsem0, xsem1, asem0, asem1):
    cid = lax.axis_index("c")
    sid = lax.axis_index("s")
    wid = sid * NCORES + cid

    xbufs = ((xb00, xb01, xb02, xb03), (xb10, xb11, xb12, xb13))
    auxs = (aux0, aux1)
    m_shs = (m_sh0, m_sh1, m_sh2, m_sh3)
    isems = (isem0, isem1)
    xsems, asems = (xsem0, xsem1), (asem0, asem1)

    lane = lax.iota(jnp.int32, 16)
    zeros16 = jnp.zeros((16,), jnp.float32)
    rows_per_sub = ACC_ROWS // 16

    def zero_body(r, _):
        for i in range(128 // 16):
            for k in range(KSPLIT):
                xbufs[0][k][r, pl.ds(i * 16, 16)] = zeros16
            aux0[r, pl.ds(i * 16, 16)] = zeros16
            aux1[r, pl.ds(i * 16, 16)] = zeros16
        return 0
    lax.fori_loop(0, CHUNK, zero_body, 0)
    sl = pl.ds(sid * rows_per_sub, rows_per_sub)
    for k in range(KSPLIT):
        pltpu.sync_copy(xbufs[0][k], m_shs[k].at[sl])
    pltpu.sync_copy(aux0, a_sh.at[sl])

    one_vec = jnp.where(lane == 0, 1.0, 0.0).astype(jnp.float32)

    def one_body(r, _):
        aux0[r, pl.ds(16, 16)] = one_vec
        aux1[r, pl.ds(16, 16)] = one_vec
        return 0
    lax.fori_loop(0, CHUNK, one_body, 0)

    pltpu.sync_copy(tgt_hbm.at[wid], tgt)
    plsc.subcore_barrier()

    def start_in(j, b):
        base = wid * ROWS_PER_WORKER + j * CHUNK
        return [
            pltpu.async_copy(
                input_hbm.at[pl.ds(base, CHUNK), pl.ds(k * 128, 128)],
                xbufs[b][k], isems[b])
            for k in range(KSPLIT)
        ]

    pend = {0: start_in(0, 0)}
    scat = {}
    for j in range(NCHUNKS):
        b = j % 2
        if j >= 1:
            for d in scat.pop(j - 1):
                d.wait()
        if j + 1 < NCHUNKS:
            pend[j + 1] = start_in(j + 1, 1 - b)
        for d in pend.pop(j):
            d.wait()
        xbk, aux = xbufs[b], auxs[b]

        def row_body(r, _):
            acc = zeros16
            for k in range(KSPLIT):
                for i in range(128 // 16):
                    xv = xbk[k][r, pl.ds(i * 16, 16)]
                    acc = acc + xv * xv
            aux[r, pl.ds(0, 16)] = acc
            return 0
        lax.fori_loop(0, CHUNK, row_body, 0)

        idx = tgt.at[j]
        ds = [pltpu.async_copy(xbk[k], m_shs[k].at[idx], xsems[b], add=True)
              for k in range(KSPLIT)]
        ds.append(pltpu.async_copy(aux, a_sh.at[idx], asems[b], add=True))
        scat[j] = ds

    for d in scat.pop(NCHUNKS - 1):
        d.wait()
    plsc.subcore_barrier()
    for k in range(KSPLIT):
        pltpu.sync_copy(m_shs[k].at[sl], out_m_hbm.at[cid, k, sl])
    pltpu.sync_copy(a_sh.at[sl], out_a_hbm.at[cid, sl])


def _tc_epilogue(m_ref, a_ref, c_ref, lam_ref, o_ref):
    dot = jnp.zeros((NUM_CLASSES, 1), jnp.float32)
    for k in range(KSPLIT):
        mk = m_ref[0, k, 0:NUM_CLASSES, :] + m_ref[1, k, 0:NUM_CLASSES, :]
        ck = c_ref[:, k * 128:(k + 1) * 128]
        dot = dot + jnp.sum(ck * mk, axis=1, keepdims=True)
    c = c_ref[...]
    cn2 = jnp.sum(c * c, axis=1, keepdims=True)
    w = a_ref[0, 0:NUM_CLASSES, :] + a_ref[1, 0:NUM_CLASSES, :]
    lane = lax.broadcasted_iota(jnp.int32, (NUM_CLASSES, AUX_W), 1)
    q = jnp.sum(jnp.where(lane < 16, w, 0.0), axis=1, keepdims=True)
    cnt = jnp.sum(jnp.where(lane == 16, w, 0.0), axis=1, keepdims=True)
    s = q - 2.0 * dot + cnt * cn2
    ratio = jnp.where(cnt > 0, s / jnp.where(cnt > 0, cnt, 1.0), 0.0)
    val = jnp.sum(ratio) * lam_ref[0] * (0.5 / BATCH)
    o_ref[...] = jnp.full((1, 1), val, jnp.float32)


def kernel(input, target, lambdas, center):
    tgt3 = target.astype(jnp.int32).reshape(NUM_WORKERS, NCHUNKS, CHUNK)

    mesh = plsc.VectorSubcoreMesh(core_axis_name="c", subcore_axis_name="s")
    m_acc, a_acc = pl.kernel(
        _sc_phase1,
        out_type=(
            jax.ShapeDtypeStruct((NCORES, KSPLIT, ACC_ROWS, 128),
                                 jnp.float32),
            jax.ShapeDtypeStruct((NCORES, ACC_ROWS, AUX_W), jnp.float32),
        ),
        mesh=mesh,
        compiler_params=pltpu.CompilerParams(needs_layout_passes=False),
        scratch_types=(
            [pltpu.VMEM((CHUNK, 128), jnp.float32) for _ in range(8)]
            + [
                pltpu.VMEM((CHUNK, AUX_W), jnp.float32),
                pltpu.VMEM((CHUNK, AUX_W), jnp.float32),
                pltpu.VMEM((NCHUNKS, CHUNK), jnp.int32),
            ]
            + [pltpu.VMEM_SHARED((ACC_ROWS, 128), jnp.float32)
               for _ in range(KSPLIT)]
            + [
                pltpu.VMEM_SHARED((ACC_ROWS, AUX_W), jnp.float32),
                pltpu.SemaphoreType.DMA,
                pltpu.SemaphoreType.DMA,
                pltpu.SemaphoreType.DMA,
                pltpu.SemaphoreType.DMA,
                pltpu.SemaphoreType.DMA,
                pltpu.SemaphoreType.DMA,
            ]
        ),
    )(input, tgt3)

    lam = jnp.asarray(lambdas, jnp.float32).reshape(1)
    out = pl.pallas_call(
        _tc_epilogue,
        out_shape=jax.ShapeDtypeStruct((1, 1), jnp.float32),
        in_specs=[
            pl.BlockSpec(memory_space=pltpu.VMEM),
            pl.BlockSpec(memory_space=pltpu.VMEM),
            pl.BlockSpec(memory_space=pltpu.VMEM),
            pl.BlockSpec(memory_space=pltpu.SMEM),
        ],
        out_specs=pl.BlockSpec(memory_space=pltpu.VMEM),
    )(m_acc, a_acc, center, lam)
    return out[0, 0]


# speedup vs baseline: 1.0622x; 1.0622x over previous
import jax
import jax.numpy as jnp
from jax import lax
from jax.experimental import pallas as pl
from jax.experimental.pallas import tpu as pltpu
from jax.experimental.pallas import tpu_sc as plsc

NUM_CLASSES = 1000
FEATURE_DIM = 512
BATCH = 16384

NCORES = 2
NUM_WORKERS = 32
ROWS_PER_WORKER = BATCH // NUM_WORKERS
CHUNK = 32
NCHUNKS = ROWS_PER_WORKER // CHUNK
ACC_ROWS = 1024
AUX_W = 128
KSPLIT = FEATURE_DIM // 128
NBUF = 4


def _sc_phase1(input_hbm, tgt_hbm, out_m_hbm, out_a_hbm, *refs):
    xbufs = tuple(tuple(refs[b * KSPLIT + k] for k in range(KSPLIT))
                  for b in range(NBUF))
    o = NBUF * KSPLIT
    auxs = tuple(refs[o + b] for b in range(NBUF))
    o += NBUF
    tgt = refs[o]
    o += 1
    m_shs = tuple(refs[o + k] for k in range(KSPLIT))
    o += KSPLIT
    a_sh = refs[o]
    o += 1
    isems = tuple(refs[o + b] for b in range(NBUF))
    o += NBUF
    ssems = tuple(refs[o + b] for b in range(NBUF))

    cid = lax.axis_index("c")
    sid = lax.axis_index("s")
    wid = sid * NCORES + cid

    lane = lax.iota(jnp.int32, 16)
    zeros16 = jnp.zeros((16,), jnp.float32)
    rows_per_sub = ACC_ROWS // 16
    sl = pl.ds(sid * rows_per_sub, rows_per_sub)

    def start_in(j, b):
        base = wid * ROWS_PER_WORKER + j * CHUNK
        return [
            pltpu.async_copy(
                input_hbm.at[pl.ds(base, CHUNK), pl.ds(k * 128, 128)],
                xbufs[b][k], isems[b])
            for k in range(KSPLIT)
        ]

    pend = {0: start_in(0, 0), 1: start_in(1, 1)}
    pltpu.sync_copy(tgt_hbm.at[wid], tgt)

    def zero_body(r, _):
        for i in range(AUX_W // 16):
            for b in range(NBUF):
                auxs[b][r, pl.ds(i * 16, 16)] = zeros16
        return 0
    lax.fori_loop(0, CHUNK, zero_body, 0)
    for k in range(KSPLIT):
        for h in range(rows_per_sub // CHUNK):
            pltpu.sync_copy(
                auxs[0],
                m_shs[k].at[pl.ds(sid * rows_per_sub + h * CHUNK, CHUNK)])
    for h in range(rows_per_sub // CHUNK):
        pltpu.sync_copy(
            auxs[0], a_sh.at[pl.ds(sid * rows_per_sub + h * CHUNK, CHUNK)])

    one_vec = jnp.where(lane == 0, 1.0, 0.0).astype(jnp.float32)

    def one_body(r, _):
        for b in range(NBUF):
            auxs[b][r, pl.ds(16, 16)] = one_vec
        return 0
    lax.fori_loop(0, CHUNK, one_body, 0)

    plsc.subcore_barrier()

    scat = {}
    for j in range(NCHUNKS):
        b = j % NBUF
        if j >= 2:
            for d in scat.pop(j - 2):
                d.wait()
        if j + 2 < NCHUNKS:
            pend[j + 2] = start_in(j + 2, (j + 2) % NBUF)
        for d in pend.pop(j):
            d.wait()
        xbk, aux = xbufs[b], auxs[b]

        def row_body(r, _):
            acc = zeros16
            for k in range(KSPLIT):
                for i in range(128 // 16):
                    xv = xbk[k][r, pl.ds(i * 16, 16)]
                    acc = acc + xv * xv
            aux[r, pl.ds(0, 16)] = acc
            return 0
        lax.fori_loop(0, CHUNK, row_body, 0)

        idx = tgt.at[j]
        ds = [pltpu.async_copy(xbk[k], m_shs[k].at[idx], ssems[b], add=True)
              for k in range(KSPLIT)]
        ds.append(pltpu.async_copy(aux, a_sh.at[idx], ssems[b], add=True))
        scat[j] = ds

    for j in (NCHUNKS - 2, NCHUNKS - 1):
        for d in scat.pop(j):
            d.wait()
    plsc.subcore_barrier()
    for k in range(KSPLIT):
        pltpu.sync_copy(m_shs[k].at[sl], out_m_hbm.at[cid, k, sl])
    pltpu.sync_copy(a_sh.at[sl], out_a_hbm.at[cid, sl])


def _tc_epilogue(m_ref, a_ref, c_ref, lam_ref, o_ref):
    dot = jnp.zeros((NUM_CLASSES, 1), jnp.float32)
    for k in range(KSPLIT):
        mk = m_ref[0, k, 0:NUM_CLASSES, :] + m_ref[1, k, 0:NUM_CLASSES, :]
        ck = c_ref[:, k * 128:(k + 1) * 128]
        dot = dot + jnp.sum(ck * mk, axis=1, keepdims=True)
    c = c_ref[...]
    cn2 = jnp.sum(c * c, axis=1, keepdims=True)
    w = a_ref[0, 0:NUM_CLASSES, :] + a_ref[1, 0:NUM_CLASSES, :]
    lane = lax.broadcasted_iota(jnp.int32, (NUM_CLASSES, AUX_W), 1)
    q = jnp.sum(jnp.where(lane < 16, w, 0.0), axis=1, keepdims=True)
    cnt = jnp.sum(jnp.where(lane == 16, w, 0.0), axis=1, keepdims=True)
    s = q - 2.0 * dot + cnt * cn2
    ratio = jnp.where(cnt > 0, s / jnp.where(cnt > 0, cnt, 1.0), 0.0)
    val = jnp.sum(ratio) * lam_ref[0] * (0.5 / BATCH)
    o_ref[...] = jnp.full((1, 1), val, jnp.float32)


def kernel(input, target, lambdas, center):
    tgt3 = target.astype(jnp.int32).reshape(NUM_WORKERS, NCHUNKS, CHUNK)

    mesh = plsc.VectorSubcoreMesh(core_axis_name="c", subcore_axis_name="s")
    m_acc, a_acc = pl.kernel(
        _sc_phase1,
        out_type=(
            jax.ShapeDtypeStruct((NCORES, KSPLIT, ACC_ROWS, 128),
                                 jnp.float32),
            jax.ShapeDtypeStruct((NCORES, ACC_ROWS, AUX_W), jnp.float32),
        ),
        mesh=mesh,
        compiler_params=pltpu.CompilerParams(needs_layout_passes=False),
        scratch_types=(
            [pltpu.VMEM((CHUNK, 128), jnp.float32)
             for _ in range(NBUF * KSPLIT)]
            + [pltpu.VMEM((CHUNK, AUX_W), jnp.float32) for _ in range(NBUF)]
            + [pltpu.VMEM((NCHUNKS, CHUNK), jnp.int32)]
            + [pltpu.VMEM_SHARED((ACC_ROWS, 128), jnp.float32)
               for _ in range(KSPLIT)]
            + [pltpu.VMEM_SHARED((ACC_ROWS, AUX_W), jnp.float32)]
            + [pltpu.SemaphoreType.DMA for _ in range(2 * NBUF)]
        ),
    )(input, tgt3)

    lam = jnp.asarray(lambdas, jnp.float32).reshape(1)
    out = pl.pallas_call(
        _tc_epilogue,
        out_shape=jax.ShapeDtypeStruct((1, 1), jnp.float32),
        in_specs=[
            pl.BlockSpec(memory_space=pltpu.VMEM),
            pl.BlockSpec(memory_space=pltpu.VMEM),
            pl.BlockSpec(memory_space=pltpu.VMEM),
            pl.BlockSpec(memory_space=pltpu.SMEM),
        ],
        out_specs=pl.BlockSpec(memory_space=pltpu.VMEM),
    )(m_acc, a_acc, center, lam)
    return out[0, 0]
